# baseline (speedup 1.0000x reference)
import functools

import jax
import jax.numpy as jnp
from jax import lax
from jax.experimental import pallas as pl
from jax.experimental.pallas import tpu as pltpu
from jax.experimental.pallas import tpu_sc as plsc

B = 16384
D = 64
L = 16
_EPS = 1e-12

NUM_FILES = 1000000
NUM_TESTS = 100000

_info = plsc.get_sparse_core_info()
NC = _info.num_cores
NS = _info.num_subcores
NW = NC * NS
BPW = B // NW
NG = BPW // L
SUB = 2
NSUB = L // SUB
NBUF = 4
DEPTH = 3
TH = 8


def _rsqrt_newton(x):
    i = lax.bitcast_convert_type(x, jnp.int32)
    i = jnp.int32(0x5F3759DF) - lax.shift_right_arithmetic(i, 1)
    y = lax.bitcast_convert_type(i, jnp.float32)
    half_x = x * 0.5
    for _ in range(3):
        y = y * (1.5 - half_x * y * y)
    return y


def _make_sc_kernel():
    mesh = plsc.VectorSubcoreMesh(core_axis_name="c", subcore_axis_name="s")

    @functools.partial(
        pl.kernel,
        mesh=mesh,
        out_type=jax.ShapeDtypeStruct((B,), jnp.float32),
        compiler_params=pltpu.CompilerParams(needs_layout_passes=False),
        scratch_types=[
            pltpu.VMEM((BPW,), jnp.int32),
            pltpu.VMEM((BPW + L,), jnp.int32),
            pltpu.VMEM((NBUF, SUB, 8, 8, 128), jnp.float32),
            pltpu.VMEM((2, TH, 128), jnp.float32),
            pltpu.VMEM((BPW // 2, 2 * D), jnp.float32),
            pltpu.VMEM((BPW,), jnp.float32),
            pltpu.SemaphoreType.DMA,
            pltpu.SemaphoreType.DMA,
            pltpu.SemaphoreType.DMA,
            pltpu.SemaphoreType.DMA,
            pltpu.SemaphoreType.DMA,
            pltpu.SemaphoreType.DMA,
        ],
    )
    def sc_kernel(fidx_hbm, tidx_hbm, ftabd_hbm, ttab2_hbm, out_hbm,
                  fidx_v, tidx_v, fblk_v, tbuf_v, frows_v, out_v,
                  semf0, semf1, semf2, semf3, semt0, semt1):
        semf = (semf0, semf1, semf2, semf3)
        semt = (semt0, semt1)
        wid = lax.axis_index("s") * NC + lax.axis_index("c")
        base = wid * BPW

        pltpu.sync_copy(fidx_hbm.at[pl.ds(base, BPW)], fidx_v)
        pltpu.sync_copy(tidx_hbm.at[pl.ds(base, BPW)],
                        tidx_v.at[pl.ds(0, BPW)])

        lane16 = lax.iota(jnp.int32, L)

        ivecs = [(jnp.int32(2 * k) + lax.shift_right_logical(lane16, 3))
                 for k in range(D // L)]
        svec = lane16 & 7

        def f_fire(jblk, buf, rr):
            off = pl.multiple_of(jblk * 128, 128)
            for a in range(8):
                pltpu.async_copy(
                    ftabd_hbm.at[a, slice(None), pl.ds(off, 128)],
                    fblk_v.at[buf, rr, a], semf[buf])

        def f_drain(buf, rr):
            pltpu.make_async_copy(
                ftabd_hbm.at[slice(None), slice(None), pl.ds(0, 128)],
                fblk_v.at[buf, rr], semf[buf]).wait()

        def f_body(g, _):
            fvec = fidx_v[pl.ds(g * L, L)]
            jvec = lax.shift_right_logical(fvec, 7)
            lvec = fvec & 127

            @pl.when(g == 0)
            def _():
                for m in range(DEPTH):
                    for rr in range(SUB):
                        f_fire(jvec[m * SUB + rr], m, rr)

            for q in range(NSUB):
                buf = q % NBUF
                nbuf = (q + DEPTH) % NBUF
                if q + DEPTH < NSUB:
                    for rr in range(SUB):
                        f_fire(jvec[(q + DEPTH) * SUB + rr], nbuf, rr)
                else:
                    @pl.when(g + 1 < NG)
                    def _():
                        nvec = lax.shift_right_logical(
                            fidx_v[pl.ds((g + 1) * L, L)], 7)
                        for rr in range(SUB):
                            f_fire(nvec[(q + DEPTH - NSUB) * SUB + rr],
                                   nbuf, rr)

                for rr in range(SUB):
                    f_drain(buf, rr)
                for rr in range(SUB):
                    j = q * SUB + rr
                    lanev = jnp.full((L,), lvec[j], jnp.int32)
                    for k in range(D // L):
                        v = plsc.load_gather(
                            fblk_v,
                            [jnp.full((L,), buf, jnp.int32),
                             jnp.full((L,), rr, jnp.int32),
                             ivecs[k], svec, lanev])
                        frows_v[g * 8 + q, pl.ds(rr * D + k * L, L)] = v
            return 0

        lax.fori_loop(0, NG, f_body, 0)

        def t_fire(h, buf):
            tvec = lax.shift_right_logical(tidx_v[pl.ds(h * TH, L)], 1)
            for j in range(TH):
                pltpu.async_copy(
                    ttab2_hbm.at[tvec[j]], tbuf_v.at[buf, j], semt[buf])

        def t_drain(buf):
            pltpu.make_async_copy(
                ttab2_hbm.at[pl.ds(0, TH)], tbuf_v.at[buf], semt[buf]).wait()

        t_fire(jnp.int32(0), 0)

        def g_body(g, _):
            acc_dot = jnp.zeros((L,), jnp.float32)
            acc_nf = jnp.zeros((L,), jnp.float32)
            acc_nt = jnp.zeros((L,), jnp.float32)
            for hh in range(2):
                h = g * 2 + hh
                buf = hh
                nbuf = 1 - hh

                @pl.when(h + 1 < 2 * NG)
                def _():
                    t_fire(h + 1, nbuf)

                t_drain(buf)
                tvec = (tidx_v[pl.ds(h * TH, L)] & 1) * D
                for j in range(TH):
                    ts = tvec[j]
                    frow = h * (TH // 2) + (j >> 1)
                    fcol = (j & 1) * D
                    f = [frows_v[frow, pl.ds(fcol + k * L, L)]
                         for k in range(D // L)]
                    t = [tbuf_v[buf, j, pl.ds(ts + k * L, L)]
                         for k in range(D // L)]
                    p_dot = f[0] * t[0]
                    p_nf = f[0] * f[0]
                    p_nt = t[0] * t[0]
                    for k in range(1, D // L):
                        p_dot = p_dot + f[k] * t[k]
                        p_nf = p_nf + f[k] * f[k]
                        p_nt = p_nt + t[k] * t[k]
                    m = lane16 == (hh * TH + j)
                    acc_dot = jnp.where(m, jnp.sum(p_dot), acc_dot)
                    acc_nf = jnp.where(m, jnp.sum(p_nf), acc_nf)
                    acc_nt = jnp.where(m, jnp.sum(p_nt), acc_nt)
            denom = jnp.maximum(acc_nf, _EPS) * jnp.maximum(acc_nt, _EPS)
            out_v[pl.ds(g * L, L)] = acc_dot * _rsqrt_newton(denom)
            return 0

        lax.fori_loop(0, NG, g_body, 0)

        pltpu.sync_copy(out_v, out_hbm.at[pl.ds(base, BPW)])

    return sc_kernel


_sc_kernel = _make_sc_kernel()


@jax.jit
def kernel(file, test, file_table, test_table):
    ftabd = file_table.T.reshape(8, 8, NUM_FILES)
    ttab2 = test_table.reshape(NUM_TESTS // 2, 2 * D)
    out = _sc_kernel(file.reshape(B), test.reshape(B), ftabd, ttab2)
    return out.reshape(B, 1)


# speedup vs baseline: 1.0695x; 1.0695x over previous
import functools

import jax
import jax.numpy as jnp
from jax import lax
from jax.experimental import pallas as pl
from jax.experimental.pallas import tpu as pltpu
from jax.experimental.pallas import tpu_sc as plsc

B = 16384
D = 64
L = 16
_EPS = 1e-12

NUM_FILES = 1000000
NUM_TESTS = 100000

_info = plsc.get_sparse_core_info()
NC = _info.num_cores
NS = _info.num_subcores
NW = NC * NS
BPW = B // NW
NG = BPW // L
SUB = 2
NSUB = L // SUB
NBUF = 4
DEPTH = 3
TH = 8


def _rsqrt_newton(x):
    i = lax.bitcast_convert_type(x, jnp.int32)
    i = jnp.int32(0x5F3759DF) - lax.shift_right_arithmetic(i, 1)
    y = lax.bitcast_convert_type(i, jnp.float32)
    half_x = x * 0.5
    for _ in range(3):
        y = y * (1.5 - half_x * y * y)
    return y


def _make_sc_kernel():
    mesh = plsc.VectorSubcoreMesh(core_axis_name="c", subcore_axis_name="s")

    @functools.partial(
        pl.kernel,
        mesh=mesh,
        out_type=jax.ShapeDtypeStruct((B,), jnp.float32),
        compiler_params=pltpu.CompilerParams(needs_layout_passes=False),
        scratch_types=[
            pltpu.VMEM((BPW,), jnp.int32),
            pltpu.VMEM((BPW + L,), jnp.int32),
            pltpu.VMEM((NBUF, SUB, 8, 8, 128), jnp.float32),
            pltpu.VMEM((2, TH, 8, D), jnp.float32),
            pltpu.VMEM((BPW // 2, 2 * D), jnp.float32),
            pltpu.VMEM((BPW,), jnp.float32),
            pltpu.SemaphoreType.DMA,
            pltpu.SemaphoreType.DMA,
            pltpu.SemaphoreType.DMA,
            pltpu.SemaphoreType.DMA,
            pltpu.SemaphoreType.DMA,
            pltpu.SemaphoreType.DMA,
        ],
    )
    def sc_kernel(fidx_hbm, tidx_hbm, ftabd_hbm, ttab3_hbm, out_hbm,
                  fidx_v, tidx_v, fblk_v, tbuf_v, frows_v, out_v,
                  semf0, semf1, semf2, semf3, semt0, semt1):
        semf = (semf0, semf1, semf2, semf3)
        semt = (semt0, semt1)
        wid = lax.axis_index("s") * NC + lax.axis_index("c")
        base = wid * BPW

        pltpu.sync_copy(fidx_hbm.at[pl.ds(base, BPW)], fidx_v)
        pltpu.sync_copy(tidx_hbm.at[pl.ds(base, BPW)],
                        tidx_v.at[pl.ds(0, BPW)])

        lane16 = lax.iota(jnp.int32, L)

        ivecs = [(jnp.int32(2 * k) + lax.shift_right_logical(lane16, 3))
                 for k in range(D // L)]
        svec = lane16 & 7

        def f_fire(jblk, buf, rr):
            off = pl.multiple_of(jblk * 128, 128)
            for a in range(8):
                pltpu.async_copy(
                    ftabd_hbm.at[a, slice(None), pl.ds(off, 128)],
                    fblk_v.at[buf, rr, a], semf[buf])

        def f_drain(buf, rr):
            pltpu.make_async_copy(
                ftabd_hbm.at[slice(None), slice(None), pl.ds(0, 128)],
                fblk_v.at[buf, rr], semf[buf]).wait()

        def f_body(g, _):
            fvec = fidx_v[pl.ds(g * L, L)]
            jvec = lax.shift_right_logical(fvec, 7)
            lvec = fvec & 127

            @pl.when(g == 0)
            def _():
                for m in range(DEPTH):
                    for rr in range(SUB):
                        f_fire(jvec[m * SUB + rr], m, rr)

            for q in range(NSUB):
                buf = q % NBUF
                nbuf = (q + DEPTH) % NBUF
                if q + DEPTH < NSUB:
                    for rr in range(SUB):
                        f_fire(jvec[(q + DEPTH) * SUB + rr], nbuf, rr)
                else:
                    @pl.when(g + 1 < NG)
                    def _():
                        nvec = lax.shift_right_logical(
                            fidx_v[pl.ds((g + 1) * L, L)], 7)
                        for rr in range(SUB):
                            f_fire(nvec[(q + DEPTH - NSUB) * SUB + rr],
                                   nbuf, rr)

                for rr in range(SUB):
                    f_drain(buf, rr)
                for rr in range(SUB):
                    j = q * SUB + rr
                    lanev = jnp.full((L,), lvec[j], jnp.int32)
                    for k in range(D // L):
                        v = plsc.load_gather(
                            fblk_v,
                            [jnp.full((L,), buf, jnp.int32),
                             jnp.full((L,), rr, jnp.int32),
                             ivecs[k], svec, lanev])
                        frows_v[g * 8 + q, pl.ds(rr * D + k * L, L)] = v
            return 0

        lax.fori_loop(0, NG, f_body, 0)

        def t_fire(h, buf):
            tvec = lax.shift_right_logical(tidx_v[pl.ds(h * TH, L)], 3)
            for j in range(TH):
                pltpu.async_copy(
                    ttab3_hbm.at[tvec[j]], tbuf_v.at[buf, j], semt[buf])

        def t_drain(buf):
            pltpu.make_async_copy(
                ttab3_hbm.at[pl.ds(0, TH)], tbuf_v.at[buf], semt[buf]).wait()

        t_fire(jnp.int32(0), 0)

        def g_body(g, _):
            acc_dot = jnp.zeros((L,), jnp.float32)
            acc_nf = jnp.zeros((L,), jnp.float32)
            acc_nt = jnp.zeros((L,), jnp.float32)
            for hh in range(2):
                h = g * 2 + hh
                buf = hh
                nbuf = 1 - hh

                @pl.when(h + 1 < 2 * NG)
                def _():
                    t_fire(h + 1, nbuf)

                t_drain(buf)
                tvec = tidx_v[pl.ds(h * TH, L)] & 7
                for j in range(TH):
                    ts = tvec[j]
                    frow = h * (TH // 2) + (j >> 1)
                    fcol = (j & 1) * D
                    f = [frows_v[frow, pl.ds(fcol + k * L, L)]
                         for k in range(D // L)]
                    t = [tbuf_v[buf, j, ts, pl.ds(k * L, L)]
                         for k in range(D // L)]
                    p_dot = f[0] * t[0]
                    p_nf = f[0] * f[0]
                    p_nt = t[0] * t[0]
                    for k in range(1, D // L):
                        p_dot = p_dot + f[k] * t[k]
                        p_nf = p_nf + f[k] * f[k]
                        p_nt = p_nt + t[k] * t[k]
                    m = lane16 == (hh * TH + j)
                    acc_dot = jnp.where(m, jnp.sum(p_dot), acc_dot)
                    acc_nf = jnp.where(m, jnp.sum(p_nf), acc_nf)
                    acc_nt = jnp.where(m, jnp.sum(p_nt), acc_nt)
            denom = jnp.maximum(acc_nf, _EPS) * jnp.maximum(acc_nt, _EPS)
            out_v[pl.ds(g * L, L)] = acc_dot * _rsqrt_newton(denom)
            return 0

        lax.fori_loop(0, NG, g_body, 0)

        pltpu.sync_copy(out_v, out_hbm.at[pl.ds(base, BPW)])

    return sc_kernel


_sc_kernel = _make_sc_kernel()


@jax.jit
def kernel(file, test, file_table, test_table):
    ftabd = file_table.T.reshape(8, 8, NUM_FILES)
    ttab3 = test_table.reshape(NUM_TESTS // 8, 8, D)
    out = _sc_kernel(file.reshape(B), test.reshape(B), ftabd, ttab3)
    return out.reshape(B, 1)


# speedup vs baseline: 1.1398x; 1.0657x over previous
import functools

import jax
import jax.numpy as jnp
from jax import lax
from jax.experimental import pallas as pl
from jax.experimental.pallas import tpu as pltpu
from jax.experimental.pallas import tpu_sc as plsc

B = 16384
D = 64
L = 16
_EPS = 1e-12

NUM_FILES = 1000000
NUM_TESTS = 100000

_info = plsc.get_sparse_core_info()
NC = _info.num_cores
NS = _info.num_subcores
NW = NC * NS
BPW = B // NW
NG = BPW // L
SUB = 1
NSUB = L // SUB
NBUF = 8
DEPTH = 7
TH = 8


def _rsqrt_newton(x):
    i = lax.bitcast_convert_type(x, jnp.int32)
    i = jnp.int32(0x5F3759DF) - lax.shift_right_arithmetic(i, 1)
    y = lax.bitcast_convert_type(i, jnp.float32)
    half_x = x * 0.5
    for _ in range(3):
        y = y * (1.5 - half_x * y * y)
    return y


def _make_sc_kernel():
    mesh = plsc.VectorSubcoreMesh(core_axis_name="c", subcore_axis_name="s")

    @functools.partial(
        pl.kernel,
        mesh=mesh,
        out_type=jax.ShapeDtypeStruct((B,), jnp.float32),
        compiler_params=pltpu.CompilerParams(needs_layout_passes=False),
        scratch_types=[
            pltpu.VMEM((BPW,), jnp.int32),
            pltpu.VMEM((BPW + L,), jnp.int32),
            pltpu.VMEM((NBUF, SUB, 8, 8, 128), jnp.float32),
            pltpu.VMEM((2, TH, 8, D), jnp.float32),
            pltpu.VMEM((BPW // 2, 2 * D), jnp.float32),
            pltpu.VMEM((BPW,), jnp.float32),
            pltpu.SemaphoreType.DMA,
            pltpu.SemaphoreType.DMA,
            pltpu.SemaphoreType.DMA,
            pltpu.SemaphoreType.DMA,
            pltpu.SemaphoreType.DMA,
            pltpu.SemaphoreType.DMA,
            pltpu.SemaphoreType.DMA,
            pltpu.SemaphoreType.DMA,
            pltpu.SemaphoreType.DMA,
            pltpu.SemaphoreType.DMA,
        ],
    )
    def sc_kernel(fidx_hbm, tidx_hbm, ftabd_hbm, ttab3_hbm, out_hbm,
                  fidx_v, tidx_v, fblk_v, tbuf_v, frows_v, out_v,
                  semf0, semf1, semf2, semf3, semf4, semf5, semf6, semf7,
                  semt0, semt1):
        semf = (semf0, semf1, semf2, semf3, semf4, semf5, semf6, semf7)
        semt = (semt0, semt1)
        wid = lax.axis_index("s") * NC + lax.axis_index("c")
        base = wid * BPW

        pltpu.sync_copy(fidx_hbm.at[pl.ds(base, BPW)], fidx_v)
        pltpu.sync_copy(tidx_hbm.at[pl.ds(base, BPW)],
                        tidx_v.at[pl.ds(0, BPW)])

        lane16 = lax.iota(jnp.int32, L)

        ivecs = [(jnp.int32(2 * k) + lax.shift_right_logical(lane16, 3))
                 for k in range(D // L)]
        svec = lane16 & 7

        def f_fire(jblk, buf, rr):
            off = pl.multiple_of(jblk * 128, 128)
            for a in range(8):
                pltpu.async_copy(
                    ftabd_hbm.at[a, slice(None), pl.ds(off, 128)],
                    fblk_v.at[buf, rr, a], semf[buf])

        def f_drain(buf, rr):
            pltpu.make_async_copy(
                ftabd_hbm.at[slice(None), slice(None), pl.ds(0, 128)],
                fblk_v.at[buf, rr], semf[buf]).wait()

        def f_body(g, _):
            fvec = fidx_v[pl.ds(g * L, L)]
            jvec = lax.shift_right_logical(fvec, 7)
            lvec = fvec & 127

            @pl.when(g == 0)
            def _():
                for m in range(DEPTH):
                    for rr in range(SUB):
                        f_fire(jvec[m * SUB + rr], m, rr)

            for q in range(NSUB):
                buf = q % NBUF
                nbuf = (q + DEPTH) % NBUF
                if q + DEPTH < NSUB:
                    for rr in range(SUB):
                        f_fire(jvec[(q + DEPTH) * SUB + rr], nbuf, rr)
                else:
                    @pl.when(g + 1 < NG)
                    def _():
                        nvec = lax.shift_right_logical(
                            fidx_v[pl.ds((g + 1) * L, L)], 7)
                        for rr in range(SUB):
                            f_fire(nvec[(q + DEPTH - NSUB) * SUB + rr],
                                   nbuf, rr)

                for rr in range(SUB):
                    f_drain(buf, rr)
                for rr in range(SUB):
                    j = q * SUB + rr
                    lanev = jnp.full((L,), lvec[j], jnp.int32)
                    for k in range(D // L):
                        v = plsc.load_gather(
                            fblk_v,
                            [jnp.full((L,), buf, jnp.int32),
                             jnp.full((L,), rr, jnp.int32),
                             ivecs[k], svec, lanev])
                        frows_v[g * 8 + (j >> 1),
                                pl.ds((j & 1) * D + k * L, L)] = v
            return 0

        lax.fori_loop(0, NG, f_body, 0)

        def t_fire(h, buf):
            tvec = lax.shift_right_logical(tidx_v[pl.ds(h * TH, L)], 3)
            for j in range(TH):
                pltpu.async_copy(
                    ttab3_hbm.at[tvec[j]], tbuf_v.at[buf, j], semt[buf])

        def t_drain(buf):
            pltpu.make_async_copy(
                ttab3_hbm.at[pl.ds(0, TH)], tbuf_v.at[buf], semt[buf]).wait()

        t_fire(jnp.int32(0), 0)

        def g_body(g, _):
            acc_dot = jnp.zeros((L,), jnp.float32)
            acc_nf = jnp.zeros((L,), jnp.float32)
            acc_nt = jnp.zeros((L,), jnp.float32)
            for hh in range(2):
                h = g * 2 + hh
                buf = hh
                nbuf = 1 - hh

                @pl.when(h + 1 < 2 * NG)
                def _():
                    t_fire(h + 1, nbuf)

                t_drain(buf)
                tvec = tidx_v[pl.ds(h * TH, L)] & 7
                for j in range(TH):
                    ts = tvec[j]
                    frow = h * (TH // 2) + (j >> 1)
                    fcol = (j & 1) * D
                    f = [frows_v[frow, pl.ds(fcol + k * L, L)]
                         for k in range(D // L)]
                    t = [tbuf_v[buf, j, ts, pl.ds(k * L, L)]
                         for k in range(D // L)]
                    p_dot = f[0] * t[0]
                    p_nf = f[0] * f[0]
                    p_nt = t[0] * t[0]
                    for k in range(1, D // L):
                        p_dot = p_dot + f[k] * t[k]
                        p_nf = p_nf + f[k] * f[k]
                        p_nt = p_nt + t[k] * t[k]
                    m = lane16 == (hh * TH + j)
                    acc_dot = jnp.where(m, jnp.sum(p_dot), acc_dot)
                    acc_nf = jnp.where(m, jnp.sum(p_nf), acc_nf)
                    acc_nt = jnp.where(m, jnp.sum(p_nt), acc_nt)
            denom = jnp.maximum(acc_nf, _EPS) * jnp.maximum(acc_nt, _EPS)
            out_v[pl.ds(g * L, L)] = acc_dot * _rsqrt_newton(denom)
            return 0

        lax.fori_loop(0, NG, g_body, 0)

        pltpu.sync_copy(out_v, out_hbm.at[pl.ds(base, BPW)])

    return sc_kernel


_sc_kernel = _make_sc_kernel()


@jax.jit
def kernel(file, test, file_table, test_table):
    ftabd = file_table.T.reshape(8, 8, NUM_FILES)
    ttab3 = test_table.reshape(NUM_TESTS // 8, 8, D)
    out = _sc_kernel(file.reshape(B), test.reshape(B), ftabd, ttab3)
    return out.reshape(B, 1)


# speedup vs baseline: 1.1435x; 1.0033x over previous
import functools

import jax
import jax.numpy as jnp
from jax import lax
from jax.experimental import pallas as pl
from jax.experimental.pallas import tpu as pltpu
from jax.experimental.pallas import tpu_sc as plsc

B = 16384
D = 64
L = 16
_EPS = 1e-12

NUM_FILES = 1000000
NUM_TESTS = 100000

_info = plsc.get_sparse_core_info()
NC = _info.num_cores
NS = _info.num_subcores
NW = NC * NS
BPW = B // NW
NG = BPW // L
SUB = 1
NSUB = L // SUB
NBUF = 8
DEPTH = 7
TH = 8


def _rsqrt_newton(x):
    i = lax.bitcast_convert_type(x, jnp.int32)
    i = jnp.int32(0x5F3759DF) - lax.shift_right_arithmetic(i, 1)
    y = lax.bitcast_convert_type(i, jnp.float32)
    half_x = x * 0.5
    for _ in range(3):
        y = y * (1.5 - half_x * y * y)
    return y


def _make_sc_kernel():
    mesh = plsc.VectorSubcoreMesh(core_axis_name="c", subcore_axis_name="s")

    @functools.partial(
        pl.kernel,
        mesh=mesh,
        out_type=jax.ShapeDtypeStruct((B,), jnp.float32),
        compiler_params=pltpu.CompilerParams(needs_layout_passes=False),
        scratch_types=[
            pltpu.VMEM((BPW,), jnp.int32),
            pltpu.VMEM((BPW + L,), jnp.int32),
            pltpu.VMEM((NBUF, SUB, 64, 128), jnp.float32),
            pltpu.VMEM((2, TH, 8, D), jnp.float32),
            pltpu.VMEM((BPW // 2, 2 * D), jnp.float32),
            pltpu.VMEM((BPW,), jnp.float32),
            pltpu.SemaphoreType.DMA,
            pltpu.SemaphoreType.DMA,
            pltpu.SemaphoreType.DMA,
            pltpu.SemaphoreType.DMA,
            pltpu.SemaphoreType.DMA,
            pltpu.SemaphoreType.DMA,
            pltpu.SemaphoreType.DMA,
            pltpu.SemaphoreType.DMA,
            pltpu.SemaphoreType.DMA,
            pltpu.SemaphoreType.DMA,
        ],
    )
    def sc_kernel(fidx_hbm, tidx_hbm, ftabd_hbm, ttab3_hbm, out_hbm,
                  fidx_v, tidx_v, fblk_v, tbuf_v, frows_v, out_v,
                  semf0, semf1, semf2, semf3, semf4, semf5, semf6, semf7,
                  semt0, semt1):
        semf = (semf0, semf1, semf2, semf3, semf4, semf5, semf6, semf7)
        semt = (semt0, semt1)
        wid = lax.axis_index("s") * NC + lax.axis_index("c")
        base = wid * BPW

        pltpu.sync_copy(fidx_hbm.at[pl.ds(base, BPW)], fidx_v)
        pltpu.sync_copy(tidx_hbm.at[pl.ds(base, BPW)],
                        tidx_v.at[pl.ds(0, BPW)])

        lane16 = lax.iota(jnp.int32, L)

        dvecs = [(jnp.int32(k * L) + lane16) for k in range(D // L)]

        def f_fire(jblk, buf, rr):
            off = pl.multiple_of(jblk * 128, 128)
            pltpu.async_copy(
                ftabd_hbm.at[slice(None), pl.ds(off, 128)],
                fblk_v.at[buf, rr], semf[buf])

        def f_drain(buf, rr):
            pltpu.make_async_copy(
                ftabd_hbm.at[slice(None), pl.ds(0, 128)],
                fblk_v.at[buf, rr], semf[buf]).wait()

        def f_body(g, _):
            fvec = fidx_v[pl.ds(g * L, L)]
            jvec = lax.shift_right_logical(fvec, 7)
            lvec = fvec & 127

            @pl.when(g == 0)
            def _():
                for m in range(DEPTH):
                    for rr in range(SUB):
                        f_fire(jvec[m * SUB + rr], m, rr)

            for q in range(NSUB):
                buf = q % NBUF
                nbuf = (q + DEPTH) % NBUF
                if q + DEPTH < NSUB:
                    for rr in range(SUB):
                        f_fire(jvec[(q + DEPTH) * SUB + rr], nbuf, rr)
                else:
                    @pl.when(g + 1 < NG)
                    def _():
                        nvec = lax.shift_right_logical(
                            fidx_v[pl.ds((g + 1) * L, L)], 7)
                        for rr in range(SUB):
                            f_fire(nvec[(q + DEPTH - NSUB) * SUB + rr],
                                   nbuf, rr)

                for rr in range(SUB):
                    f_drain(buf, rr)
                for rr in range(SUB):
                    j = q * SUB + rr
                    lanev = jnp.full((L,), lvec[j], jnp.int32)
                    for k in range(D // L):
                        v = plsc.load_gather(
                            fblk_v,
                            [jnp.full((L,), buf, jnp.int32),
                             jnp.full((L,), rr, jnp.int32),
                             dvecs[k], lanev])
                        frows_v[g * 8 + (j >> 1),
                                pl.ds((j & 1) * D + k * L, L)] = v
            return 0

        lax.fori_loop(0, NG, f_body, 0)

        def t_fire(h, buf):
            tvec = lax.shift_right_logical(tidx_v[pl.ds(h * TH, L)], 3)
            for j in range(TH):
                pltpu.async_copy(
                    ttab3_hbm.at[tvec[j]], tbuf_v.at[buf, j], semt[buf])

        def t_drain(buf):
            pltpu.make_async_copy(
                ttab3_hbm.at[pl.ds(0, TH)], tbuf_v.at[buf], semt[buf]).wait()

        t_fire(jnp.int32(0), 0)

        def g_body(g, _):
            acc_dot = jnp.zeros((L,), jnp.float32)
            acc_nf = jnp.zeros((L,), jnp.float32)
            acc_nt = jnp.zeros((L,), jnp.float32)
            for hh in range(2):
                h = g * 2 + hh
                buf = hh
                nbuf = 1 - hh

                @pl.when(h + 1 < 2 * NG)
                def _():
                    t_fire(h + 1, nbuf)

                t_drain(buf)
                tvec = tidx_v[pl.ds(h * TH, L)] & 7
                for j in range(TH):
                    ts = tvec[j]
                    frow = h * (TH // 2) + (j >> 1)
                    fcol = (j & 1) * D
                    f = [frows_v[frow, pl.ds(fcol + k * L, L)]
                         for k in range(D // L)]
                    t = [tbuf_v[buf, j, ts, pl.ds(k * L, L)]
                         for k in range(D // L)]
                    p_dot = f[0] * t[0]
                    p_nf = f[0] * f[0]
                    p_nt = t[0] * t[0]
                    for k in range(1, D // L):
                        p_dot = p_dot + f[k] * t[k]
                        p_nf = p_nf + f[k] * f[k]
                        p_nt = p_nt + t[k] * t[k]
                    m = lane16 == (hh * TH + j)
                    acc_dot = jnp.where(m, jnp.sum(p_dot), acc_dot)
                    acc_nf = jnp.where(m, jnp.sum(p_nf), acc_nf)
                    acc_nt = jnp.where(m, jnp.sum(p_nt), acc_nt)
            denom = jnp.maximum(acc_nf, _EPS) * jnp.maximum(acc_nt, _EPS)
            out_v[pl.ds(g * L, L)] = acc_dot * _rsqrt_newton(denom)
            return 0

        lax.fori_loop(0, NG, g_body, 0)

        pltpu.sync_copy(out_v, out_hbm.at[pl.ds(base, BPW)])

    return sc_kernel


_sc_kernel = _make_sc_kernel()


@jax.jit
def kernel(file, test, file_table, test_table):
    ftabd = file_table.T
    ttab3 = test_table.reshape(NUM_TESTS // 8, 8, D)
    out = _sc_kernel(file.reshape(B), test.reshape(B), ftabd, ttab3)
    return out.reshape(B, 1)
